# baseline (speedup 1.0000x reference)
import functools

import jax
import jax.numpy as jnp
from jax import lax
from jax.experimental import pallas as pl
from jax.experimental.pallas import tpu as pltpu
from jax.experimental.pallas import tpu_sc as plsc

_N = 32768
_B = 128
_K = 256
_TEMPERATURE = 0.5

_NT = 16
_NL = _N // _NT
_NV = _NL // 16
_CAP = 64
_PK = 512
_ROWS = _B // _NT
_TOPBUF = 1152
_SENT_IDX = 0x3FFFFFFF


def _lane():
    return lax.iota(jnp.int32, 16)


def _splat(x, dtype=jnp.int32):
    return jnp.full((16,), x, dtype)


def _popcnt(mask):
    return plsc.all_reduce_population_count(mask)


def _bcast(v, i):
    idx = (_splat(0) + i).reshape(16, 1)
    return lax.gather(
        v, idx,
        dimension_numbers=lax.GatherDimensionNumbers(
            offset_dims=(), collapsed_slice_dims=(0,), start_index_map=(0,)),
        slice_sizes=(1,),
        mode=lax.GatherScatterMode.PROMISE_IN_BOUNDS)


def _sc_body(xflat, logits, gnoise, weights, out_ws, out_idx,
             keys_v, hist, hm, hidx, cand_k, cand_i, pidx, rank16,
             pk_v, pi_v, cnt_v, cntall_v, senti_v,
             idx256, w256, gidx, xg, zeros_v, rowsums_v,
             sem,
             sh_hist, sh_cnt, sh_pk, sh_pi, sh_top, sh_sums):
    wid = lax.axis_index("s")
    lanes = _lane()
    scope = jax.named_scope

    base = wid * _NL
    with scope("p1_load"):
        pltpu.sync_copy(logits.at[pl.ds(base, _NL)], xg.at[pl.ds(0, _NL)])
        pltpu.sync_copy(gnoise.at[pl.ds(base, _NL)], xg.at[pl.ds(_NL, _NL)])
        pltpu.sync_copy(weights, w256)

    def _hzero(j, _):
        for u in range(4):
            hist[pl.ds((j * 4 + u) * 16, 16)] = _splat(0)
        return 0

    lax.fori_loop(0, 64, _hzero, 0)

    def _hidx_step(j, _):
        r = j // 8
        q = j % 8
        hidx[r, pl.ds(q * 16, 16)] = _splat(r * 128 + q * 16) + lanes
        return 0

    lax.fori_loop(0, 64, _hidx_step, 0)

    def _zero_step(j, _):
        zeros_v[pl.ds(j * 16, 16)] = _splat(0)
        return 0

    lax.fori_loop(0, 16, _zero_step, 0)
    senti_v[pl.ds(0, 16)] = _splat(_SENT_IDX)
    senti_v[pl.ds(16, 16)] = _splat(_SENT_IDX)

    @pl.when(wid == 0)
    def _():
        for q in range(4):
            pltpu.sync_copy(zeros_v, sh_hist.at[pl.ds(q * 256, 256)])

    ones = _splat(1)

    def _keys_step(j, _):
        y = xg[pl.ds(j * 16, 16)] + xg[pl.ds(_NL + j * 16, 16)]
        bits = lax.bitcast_convert_type(y, jnp.int32)
        ubits = plsc.bitcast(bits, jnp.uint32)
        key = jnp.where(bits < 0, ~ubits, ubits | jnp.uint32(0x80000000))
        keys_v[pl.ds(j * 16, 16)] = key
        digit = (key >> jnp.uint32(24)).astype(jnp.int32)
        plsc.addupdate_scatter(hist, [lanes * 256 + digit], ones)
        return 0

    with scope("p1_keys"):
        lax.fori_loop(0, _NV, _keys_step, 0)
    with scope("bar1"):
        plsc.subcore_barrier()

    prefix = _splat(0, jnp.uint32)
    rem = _splat(_K)
    for p, shift in enumerate((24, 16, 8, 0)):
        maskhi = jnp.uint32((0xFFFFFF00 << shift) & 0xFFFFFFFF)

        if p > 0:
            def _hfill(j, _, _prefix=prefix, _shift=shift, _maskhi=maskhi):
                k = keys_v[pl.ds(j * 16, 16)]
                match = (k & _maskhi) == (_prefix & _maskhi)

                @pl.when(jnp.any(match))
                def _():
                    digit = ((k >> jnp.uint32(_shift))
                             & jnp.uint32(0xFF)).astype(jnp.int32)
                    plsc.addupdate_scatter(hist, [lanes * 256 + digit],
                                           ones, mask=match)
                return 0

            with scope("p2_fill"):
                lax.fori_loop(0, _NV, _hfill, 0)

        def _hreduce(j, _):
            acc = hist[pl.ds(j * 16, 16)]
            hist[pl.ds(j * 16, 16)] = _splat(0)
            for l in range(1, 16):
                acc = acc + hist[pl.ds(l * 256 + j * 16, 16)]
                hist[pl.ds(l * 256 + j * 16, 16)] = _splat(0)
            hm[pl.ds(j * 16, 16)] = acc
            return 0

        with scope("p2_reduce"):
            lax.fori_loop(0, 16, _hreduce, 0)

        with scope("p2_merge"):
            pltpu.sync_copy(hm.at[pl.ds(0, 128)],
                            sh_hist.at[hidx.at[2 * p]], add=True)
            pltpu.sync_copy(hm.at[pl.ds(128, 128)],
                            sh_hist.at[hidx.at[2 * p + 1]], add=True)
        if p == 3:
            pltpu.sync_copy(zeros_v.at[pl.ds(0, 32)],
                            sh_pk.at[pl.ds(wid * 32, 32)])
            pltpu.sync_copy(senti_v, sh_pi.at[pl.ds(wid * 32, 32)])
        with scope("p2_bar"):
            plsc.subcore_barrier()
        with scope("p2_read"):
            pltpu.sync_copy(sh_hist.at[pl.ds(p * 256, 256)], hm)

        def _select(t, carry):
            B, sfxB, hB, above = carry
            vj = 15 - t
            v = hm[pl.ds(vj * 16, 16)]
            sfx = lax.rev(plsc.cumsum(lax.rev(v, (0,))), (0,)) + above
            cond = sfx >= rem
            cnt = _popcnt(cond)
            first = (B < 0) & (cnt > 0)
            this_b = _splat(vj * 16) + cnt - 1
            sel = jnp.maximum(cnt - 1, 0)
            B = jnp.where(first, this_b, B)
            hB = jnp.where(first, _bcast(v, sel), hB)
            sfxB = jnp.where(first, _bcast(sfx, sel), sfxB)
            above = _bcast(sfx, _splat(0))
            return B, sfxB, hB, above

        with scope("p2_select"):
            B, sfxB, hB, _ = lax.fori_loop(
                0, 16, _select,
                (_splat(-1), _splat(0), _splat(0), _splat(0)))
        rem = rem - (sfxB - hB)
        prefix = prefix | (B.astype(jnp.uint32) << jnp.uint32(shift))

    def _compact(j, cnt):
        k = keys_v[pl.ds(j * 16, 16)]
        m = k >= prefix

        @pl.when(jnp.any(m))
        def _():
            pos = cnt + plsc.cumsum(jnp.where(m, 1, 0)) - 1
            pos = jnp.minimum(pos, _CAP - 1)
            plsc.store_scatter(cand_k, [pos],
                               plsc.bitcast(k, jnp.int32), mask=m)
            gi = _splat(j * 16) + lanes + base
            plsc.store_scatter(cand_i, [pos], gi, mask=m)
        return cnt + _popcnt(m)

    with scope("p3_compact"):
        cnt = lax.fori_loop(0, _NV, _compact, _splat(0))
    cnt = jnp.minimum(cnt, _CAP)

    cnt_v[...] = cnt
    with scope("p3_cntx"):
        pltpu.sync_copy(cnt_v, sh_cnt.at[pl.ds(wid * 16, 16)])
        plsc.subcore_barrier()
        pltpu.sync_copy(sh_cnt, cntall_v)

    pbase = _splat(0)
    ntot = _splat(0)
    for t in range(_NT):
        v = cntall_v[pl.ds(t * 16, 16)]
        pbase = pbase + jnp.where(_splat(t) < wid, v, 0)
        ntot = ntot + v

    def _pidx_step(q, _):
        slot = _splat(q * 16) + lanes
        pos = jnp.where(slot < cnt, pbase + slot, _splat(_PK) + wid)
        pidx[pl.ds(q * 16, 16)] = jnp.minimum(pos, _PK + 63)
        return 0

    with scope("p3_pack"):
        lax.fori_loop(0, _CAP // 16, _pidx_step, 0)
        pltpu.sync_copy(cand_k, sh_pk.at[pidx])
        pltpu.sync_copy(cand_i, sh_pi.at[pidx])
        plsc.subcore_barrier()
        pltpu.sync_copy(sh_pk.at[pl.ds(0, _PK)], pk_v)
        pltpu.sync_copy(sh_pi.at[pl.ds(0, _PK)], pi_v)

    ntot_s = ntot[0]

    for rnd in range(2):
        vsel = wid if rnd == 0 else 16 + (15 - wid)

        @pl.when(ntot_s > vsel * 16)
        def _():
            ck = plsc.bitcast(pk_v[pl.ds(vsel * 16, 16)], jnp.uint32)
            ci = pi_v[pl.ds(vsel * 16, 16)]

            def _rank(e, r):
                ke = plsc.bitcast(
                    plsc.load_gather(pk_v, [_splat(0) + e]), jnp.uint32)
                ie = plsc.load_gather(pi_v, [_splat(0) + e])
                gt = ke > ck
                tie = (ke == ck) & (ie < ci)
                return r + gt.astype(jnp.int32) + tie.astype(jnp.int32)

            r = lax.fori_loop(0, ntot_s, _rank, _splat(0))
            rank16[...] = jnp.minimum(r, _TOPBUF - 1)
            pltpu.sync_copy(pi_v.at[pl.ds(vsel * 16, 16)],
                            sh_top.at[rank16])

    with scope("p3_rankbar"):
        plsc.subcore_barrier()

    with scope("p4_idx"):
        pltpu.sync_copy(sh_top.at[pl.ds(0, _K)], idx256)

    def _cphys_step(q, _):
        cj = idx256[pl.ds(q * 16, 16)]
        idx256[pl.ds(q * 16, 16)] = ((cj >> 7) << 10) | (cj & 127)
        return 0

    lax.fori_loop(0, _K // 16, _cphys_step, 0)

    def _gfill(j, _):
        r = j // 16
        q = j % 16
        b = wid * _ROWS + r
        rb = (b // 8) * (_N * 8) + (b % 8) * 128
        gidx[pl.ds(j * 16, 16)] = idx256[pl.ds(q * 16, 16)] + rb
        return 0

    lax.fori_loop(0, _ROWS * 16, _gfill, 0)

    with scope("p4_gidx"):
        pass
    copies = []
    for r in range(_ROWS):
        for h in range(2):
            o = r * _K + h * 128
            copies.append(pltpu.async_copy(
                xflat.at[gidx.at[pl.ds(o, 128)]],
                xg.at[pl.ds(o, 128)], sem))
    with scope("p4_gather"):
        for c in copies:
            c.wait()

    def _wsum(r, sums):
        def _acc(q, a):
            a0 = a + xg[pl.ds(r * _K + q * 32, 16)] * w256[pl.ds(q * 32, 16)]
            return a0 + (xg[pl.ds(r * _K + q * 32 + 16, 16)]
                         * w256[pl.ds(q * 32 + 16, 16)])
        acc = lax.fori_loop(0, _K // 32, _acc, jnp.zeros((16,), jnp.float32))
        s = jnp.sum(acc)
        return jnp.where(lanes == r, jnp.full((16,), s, jnp.float32), sums)

    with scope("p4_wsum"):
        sums = lax.fori_loop(0, _ROWS, _wsum, jnp.zeros((16,), jnp.float32))
    rowsums_v[...] = sums
    with scope("p4_out"):
        pltpu.sync_copy(rowsums_v.at[pl.ds(0, _ROWS)],
                        sh_sums.at[pl.ds(wid * _ROWS, _ROWS)])
        plsc.subcore_barrier()

    @pl.when(wid == 0)
    def _():
        pltpu.sync_copy(sh_sums, out_ws)
        pltpu.sync_copy(sh_top.at[pl.ds(0, _K)], out_idx)


@functools.partial(jax.jit, static_argnums=())
def _run(xflat, logits, gnoise, weights):
    mesh = plsc.VectorSubcoreMesh(
        core_axis_name="c", subcore_axis_name="s", num_cores=1)
    kern = pl.kernel(
        _sc_body,
        out_type=(
            jax.ShapeDtypeStruct((_B,), jnp.float32),
            jax.ShapeDtypeStruct((_K,), jnp.int32),
        ),
        mesh=mesh,
        compiler_params=pltpu.CompilerParams(needs_layout_passes=False),
        scratch_types=[
            pltpu.VMEM((_NL,), jnp.uint32),
            pltpu.VMEM((16 * 256,), jnp.int32),
            pltpu.VMEM((256,), jnp.int32),
            pltpu.VMEM((8, 128), jnp.int32),
            pltpu.VMEM((_CAP,), jnp.int32),
            pltpu.VMEM((_CAP,), jnp.int32),
            pltpu.VMEM((_CAP,), jnp.int32),
            pltpu.VMEM((16,), jnp.int32),
            pltpu.VMEM((_PK,), jnp.int32),
            pltpu.VMEM((_PK,), jnp.int32),
            pltpu.VMEM((16,), jnp.int32),
            pltpu.VMEM((256,), jnp.int32),
            pltpu.VMEM((32,), jnp.int32),
            pltpu.VMEM((_K,), jnp.int32),
            pltpu.VMEM((_K,), jnp.float32),
            pltpu.VMEM((_ROWS * _K,), jnp.int32),
            pltpu.VMEM((2 * _NL,), jnp.float32),
            pltpu.VMEM((256,), jnp.int32),
            pltpu.VMEM((16,), jnp.float32),
            pltpu.SemaphoreType.DMA,
            pltpu.VMEM_SHARED((1024,), jnp.int32),
            pltpu.VMEM_SHARED((256,), jnp.int32),
            pltpu.VMEM_SHARED((_PK + 64,), jnp.int32),
            pltpu.VMEM_SHARED((_PK + 64,), jnp.int32),
            pltpu.VMEM_SHARED((_TOPBUF,), jnp.int32),
            pltpu.VMEM_SHARED((_B,), jnp.float32),
        ],
    )
    return kern(xflat, logits, gnoise, weights)


def kernel(x, logits, output_weights):
    gkey = jax.random.key(42)
    u = jax.random.uniform(gkey, (1, _N), dtype=jnp.float32)
    eps = 1e-20
    g = (-jnp.log(-jnp.log(u + eps) + eps)).reshape(-1)
    xp = x.reshape(16, 8, 256, 128).transpose(0, 2, 1, 3).reshape(-1)
    ws, idx = _run(xp, logits, g, output_weights)
    return (ws.reshape(_B, 1), idx.reshape(1, _K), output_weights)


# speedup vs baseline: 1.2651x; 1.2651x over previous
import functools

import jax
import jax.numpy as jnp
from jax import lax
from jax.experimental import pallas as pl
from jax.experimental.pallas import tpu as pltpu
from jax.experimental.pallas import tpu_sc as plsc

_N = 32768
_B = 128
_K = 256
_TEMPERATURE = 0.5

_NT = 16
_NL = _N // _NT
_NV = _NL // 16
_CAP = 64
_PK = 512
_ROWS = _B // _NT
_TOPBUF = 1152
_SENT_IDX = 0x3FFFFFFF


def _lane():
    return lax.iota(jnp.int32, 16)


def _splat(x, dtype=jnp.int32):
    return jnp.full((16,), x, dtype)


def _popcnt(mask):
    return plsc.all_reduce_population_count(mask)


def _bcast(v, i):
    idx = (_splat(0) + i).reshape(16, 1)
    return lax.gather(
        v, idx,
        dimension_numbers=lax.GatherDimensionNumbers(
            offset_dims=(), collapsed_slice_dims=(0,), start_index_map=(0,)),
        slice_sizes=(1,),
        mode=lax.GatherScatterMode.PROMISE_IN_BOUNDS)


def _sc_body(xflat, logits, gnoise, weights, out_ws, out_idx,
             keys_v, hist, hm, hidx, cand_k, cand_i, pidx, rank16,
             pk_v, pi_v, cnt_v, cntall_v, senti_v,
             idx256, w256, gidx, xg, zeros_v, rowsums_v,
             sem,
             sh_hist, sh_cnt, sh_pk, sh_pi, sh_top, sh_sums):
    wid = lax.axis_index("s")
    lanes = _lane()
    scope = jax.named_scope

    base = wid * _NL
    with scope("p1_load"):
        in_copies = [
            pltpu.async_copy(logits.at[pl.ds(base, _NL)],
                             xg.at[pl.ds(0, _NL)], sem),
            pltpu.async_copy(gnoise.at[pl.ds(base, _NL)],
                             xg.at[pl.ds(_NL, _NL)], sem),
            pltpu.async_copy(weights, w256, sem),
        ]

    def _hzero(j, _):
        for u in range(4):
            hist[pl.ds((j * 4 + u) * 16, 16)] = _splat(0)
        return 0

    lax.fori_loop(0, 64, _hzero, 0)

    def _hidx_step(j, _):
        r = j // 8
        q = j % 8
        hidx[r, pl.ds(q * 16, 16)] = _splat(r * 128 + q * 16) + lanes
        return 0

    lax.fori_loop(0, 64, _hidx_step, 0)

    def _zero_step(j, _):
        zeros_v[pl.ds(j * 16, 16)] = _splat(0)
        return 0

    lax.fori_loop(0, 16, _zero_step, 0)
    senti_v[pl.ds(0, 16)] = _splat(_SENT_IDX)
    senti_v[pl.ds(16, 16)] = _splat(_SENT_IDX)

    @pl.when(wid == 0)
    def _():
        for q in range(4):
            pltpu.sync_copy(zeros_v, sh_hist.at[pl.ds(q * 256, 256)])

    ones = _splat(1)
    with scope("p1_wait"):
        for c in in_copies:
            c.wait()

    def _keys_step(j, _):
        for u in range(2):
            o = (j * 2 + u) * 16
            y = xg[pl.ds(o, 16)] + xg[pl.ds(_NL + o, 16)]
            bits = lax.bitcast_convert_type(y, jnp.int32)
            ubits = plsc.bitcast(bits, jnp.uint32)
            key = jnp.where(bits < 0, ~ubits, ubits | jnp.uint32(0x80000000))
            keys_v[pl.ds(o, 16)] = key
            digit = (key >> jnp.uint32(24)).astype(jnp.int32)
            plsc.addupdate_scatter(hist, [lanes * 256 + digit], ones)
        return 0

    with scope("p1_keys"):
        lax.fori_loop(0, _NV // 2, _keys_step, 0)
    with scope("bar1"):
        plsc.subcore_barrier()

    prefix = _splat(0, jnp.uint32)
    rem = _splat(_K)
    for p, shift in enumerate((24, 16, 8, 0)):
        maskhi = jnp.uint32((0xFFFFFF00 << shift) & 0xFFFFFFFF)

        if p > 0:
            def _hfill(j, _, _prefix=prefix, _shift=shift, _maskhi=maskhi):
                pref = _prefix & _maskhi
                ms = []
                hit = None
                for u in range(8):
                    k = keys_v[pl.ds((j * 8 + u) * 16, 16)]
                    m = (k & _maskhi) == pref
                    ms.append(m)
                    hit = m if hit is None else (hit | m)

                @pl.when(jnp.any(hit))
                def _():
                    for u in range(8):
                        k = keys_v[pl.ds((j * 8 + u) * 16, 16)]
                        digit = ((k >> jnp.uint32(_shift))
                                 & jnp.uint32(0xFF)).astype(jnp.int32)
                        plsc.addupdate_scatter(hist, [lanes * 256 + digit],
                                               ones, mask=ms[u])
                return 0

            with scope("p2_fill"):
                lax.fori_loop(0, _NV // 8, _hfill, 0)

        def _hreduce(j, _):
            acc = hist[pl.ds(j * 16, 16)]
            hist[pl.ds(j * 16, 16)] = _splat(0)
            for l in range(1, 16):
                acc = acc + hist[pl.ds(l * 256 + j * 16, 16)]
                hist[pl.ds(l * 256 + j * 16, 16)] = _splat(0)
            hm[pl.ds(j * 16, 16)] = acc
            return 0

        with scope("p2_reduce"):
            lax.fori_loop(0, 16, _hreduce, 0)

        with scope("p2_merge"):
            pltpu.sync_copy(hm.at[pl.ds(0, 128)],
                            sh_hist.at[hidx.at[2 * p]], add=True)
            pltpu.sync_copy(hm.at[pl.ds(128, 128)],
                            sh_hist.at[hidx.at[2 * p + 1]], add=True)
        if p == 3:
            pltpu.sync_copy(zeros_v.at[pl.ds(0, 32)],
                            sh_pk.at[pl.ds(wid * 32, 32)])
            pltpu.sync_copy(senti_v, sh_pi.at[pl.ds(wid * 32, 32)])
        with scope("p2_bar"):
            plsc.subcore_barrier()
        with scope("p2_read"):
            pltpu.sync_copy(sh_hist.at[pl.ds(p * 256, 256)], hm)

        def _select(t, carry):
            B, sfxB, hB, above = carry
            vj = 15 - t
            v = hm[pl.ds(vj * 16, 16)]
            sfx = lax.rev(plsc.cumsum(lax.rev(v, (0,))), (0,)) + above
            cond = sfx >= rem
            cnt = _popcnt(cond)
            first = (B < 0) & (cnt > 0)
            this_b = _splat(vj * 16) + cnt - 1
            sel = jnp.maximum(cnt - 1, 0)
            B = jnp.where(first, this_b, B)
            hB = jnp.where(first, _bcast(v, sel), hB)
            sfxB = jnp.where(first, _bcast(sfx, sel), sfxB)
            above = _bcast(sfx, _splat(0))
            return B, sfxB, hB, above

        with scope("p2_select"):
            B, sfxB, hB, _ = lax.fori_loop(
                0, 16, _select,
                (_splat(-1), _splat(0), _splat(0), _splat(0)))
        rem = rem - (sfxB - hB)
        prefix = prefix | (B.astype(jnp.uint32) << jnp.uint32(shift))

    def _compact(j, cnt):
        kms = []
        hit = None
        for u in range(8):
            k = keys_v[pl.ds((j * 8 + u) * 16, 16)]
            m = k >= prefix
            kms.append((k, m))
            hit = m if hit is None else (hit | m)

        @pl.when(jnp.any(hit))
        def _():
            c = cnt
            for u in range(8):
                k, m = kms[u]
                pos = c + plsc.cumsum(jnp.where(m, 1, 0)) - 1
                pos = jnp.minimum(pos, _CAP - 1)
                plsc.store_scatter(cand_k, [pos],
                                   plsc.bitcast(k, jnp.int32), mask=m)
                gi = _splat((j * 8 + u) * 16) + lanes + base
                plsc.store_scatter(cand_i, [pos], gi, mask=m)
                c = c + _popcnt(m)

        for u in range(8):
            cnt = cnt + _popcnt(kms[u][1])
        return cnt

    with scope("p3_compact"):
        cnt = lax.fori_loop(0, _NV // 8, _compact, _splat(0))
    cnt = jnp.minimum(cnt, _CAP)

    cnt_v[...] = cnt
    with scope("p3_cntx"):
        pltpu.sync_copy(cnt_v, sh_cnt.at[pl.ds(wid * 16, 16)])
        plsc.subcore_barrier()
        pltpu.sync_copy(sh_cnt, cntall_v)

    pbase = _splat(0)
    ntot = _splat(0)
    for t in range(_NT):
        v = cntall_v[pl.ds(t * 16, 16)]
        pbase = pbase + jnp.where(_splat(t) < wid, v, 0)
        ntot = ntot + v

    def _pidx_step(q, _):
        slot = _splat(q * 16) + lanes
        pos = jnp.where(slot < cnt, pbase + slot, _splat(_PK) + wid)
        pidx[pl.ds(q * 16, 16)] = jnp.minimum(pos, _PK + 63)
        return 0

    with scope("p3_pack"):
        lax.fori_loop(0, _CAP // 16, _pidx_step, 0)
        pltpu.sync_copy(cand_k, sh_pk.at[pidx])
        pltpu.sync_copy(cand_i, sh_pi.at[pidx])
        plsc.subcore_barrier()
        pltpu.sync_copy(sh_pk.at[pl.ds(0, _PK)], pk_v)
        pltpu.sync_copy(sh_pi.at[pl.ds(0, _PK)], pi_v)

    ntot_s = ntot[0]

    with scope("p3_rank"):
        for rnd in range(2):
            vsel = wid if rnd == 0 else 16 + (15 - wid)

            @pl.when(ntot_s > vsel * 16)
            def _():
                ck = plsc.bitcast(pk_v[pl.ds(vsel * 16, 16)], jnp.uint32)
                ci = pi_v[pl.ds(vsel * 16, 16)]

                def _rank(e, r):
                    ke = plsc.bitcast(
                        plsc.load_gather(pk_v, [_splat(0) + e]), jnp.uint32)
                    ie = plsc.load_gather(pi_v, [_splat(0) + e])
                    gt = ke > ck
                    tie = (ke == ck) & (ie < ci)
                    return r + gt.astype(jnp.int32) + tie.astype(jnp.int32)

                r = lax.fori_loop(0, ntot_s, _rank, _splat(0))
                rank16[...] = jnp.minimum(r, _TOPBUF - 1)
                pltpu.sync_copy(pi_v.at[pl.ds(vsel * 16, 16)],
                                sh_top.at[rank16])

    with scope("p3_rankbar"):
        plsc.subcore_barrier()

    with scope("p4_idx"):
        pltpu.sync_copy(sh_top.at[pl.ds(0, _K)], idx256)

    def _cphys_step(q, _):
        cj = idx256[pl.ds(q * 16, 16)]
        idx256[pl.ds(q * 16, 16)] = ((cj >> 7) << 10) | (cj & 127)
        return 0

    def _gfill(j, _):
        r = j // 16
        q = j % 16
        b = wid * _ROWS + r
        rb = (b // 8) * (_N * 8) + (b % 8) * 128
        gidx[pl.ds(j * 16, 16)] = idx256[pl.ds(q * 16, 16)] + rb
        return 0

    with scope("p4_gidx"):
        lax.fori_loop(0, _K // 16, _cphys_step, 0)
        lax.fori_loop(0, _ROWS * 16, _gfill, 0)

    copies = []
    for r in range(_ROWS):
        for h in range(2):
            o = r * _K + h * 128
            copies.append(pltpu.async_copy(
                xflat.at[gidx.at[pl.ds(o, 128)]],
                xg.at[pl.ds(o, 128)], sem))
    with scope("p4_gather"):
        for c in copies:
            c.wait()

    def _wsum(r, sums):
        def _acc(q, a):
            a0 = a + xg[pl.ds(r * _K + q * 32, 16)] * w256[pl.ds(q * 32, 16)]
            return a0 + (xg[pl.ds(r * _K + q * 32 + 16, 16)]
                         * w256[pl.ds(q * 32 + 16, 16)])
        acc = lax.fori_loop(0, _K // 32, _acc, jnp.zeros((16,), jnp.float32))
        s = jnp.sum(acc)
        return jnp.where(lanes == r, jnp.full((16,), s, jnp.float32), sums)

    with scope("p4_wsum"):
        sums = lax.fori_loop(0, _ROWS, _wsum, jnp.zeros((16,), jnp.float32))
    rowsums_v[...] = sums
    with scope("p4_out"):
        pltpu.sync_copy(rowsums_v.at[pl.ds(0, _ROWS)],
                        sh_sums.at[pl.ds(wid * _ROWS, _ROWS)])
        plsc.subcore_barrier()

    @pl.when(wid == 0)
    def _():
        pltpu.sync_copy(sh_sums, out_ws)
        pltpu.sync_copy(sh_top.at[pl.ds(0, _K)], out_idx)


@functools.partial(jax.jit, static_argnums=())
def _run(xflat, logits, gnoise, weights):
    mesh = plsc.VectorSubcoreMesh(
        core_axis_name="c", subcore_axis_name="s", num_cores=1)
    kern = pl.kernel(
        _sc_body,
        out_type=(
            jax.ShapeDtypeStruct((_B,), jnp.float32),
            jax.ShapeDtypeStruct((_K,), jnp.int32),
        ),
        mesh=mesh,
        compiler_params=pltpu.CompilerParams(needs_layout_passes=False),
        scratch_types=[
            pltpu.VMEM((_NL,), jnp.uint32),
            pltpu.VMEM((16 * 256,), jnp.int32),
            pltpu.VMEM((256,), jnp.int32),
            pltpu.VMEM((8, 128), jnp.int32),
            pltpu.VMEM((_CAP,), jnp.int32),
            pltpu.VMEM((_CAP,), jnp.int32),
            pltpu.VMEM((_CAP,), jnp.int32),
            pltpu.VMEM((16,), jnp.int32),
            pltpu.VMEM((_PK,), jnp.int32),
            pltpu.VMEM((_PK,), jnp.int32),
            pltpu.VMEM((16,), jnp.int32),
            pltpu.VMEM((256,), jnp.int32),
            pltpu.VMEM((32,), jnp.int32),
            pltpu.VMEM((_K,), jnp.int32),
            pltpu.VMEM((_K,), jnp.float32),
            pltpu.VMEM((_ROWS * _K,), jnp.int32),
            pltpu.VMEM((2 * _NL,), jnp.float32),
            pltpu.VMEM((256,), jnp.int32),
            pltpu.VMEM((16,), jnp.float32),
            pltpu.SemaphoreType.DMA,
            pltpu.VMEM_SHARED((1024,), jnp.int32),
            pltpu.VMEM_SHARED((256,), jnp.int32),
            pltpu.VMEM_SHARED((_PK + 64,), jnp.int32),
            pltpu.VMEM_SHARED((_PK + 64,), jnp.int32),
            pltpu.VMEM_SHARED((_TOPBUF,), jnp.int32),
            pltpu.VMEM_SHARED((_B,), jnp.float32),
        ],
    )
    return kern(xflat, logits, gnoise, weights)


def kernel(x, logits, output_weights):
    gkey = jax.random.key(42)
    u = jax.random.uniform(gkey, (1, _N), dtype=jnp.float32)
    eps = 1e-20
    g = (-jnp.log(-jnp.log(u + eps) + eps)).reshape(-1)
    xp = x.reshape(16, 8, 256, 128).transpose(0, 2, 1, 3).reshape(-1)
    ws, idx = _run(xp, logits, g, output_weights)
    return (ws.reshape(_B, 1), idx.reshape(1, _K), output_weights)


# speedup vs baseline: 1.2892x; 1.0190x over previous
import functools

import jax
import jax.numpy as jnp
from jax import lax
from jax.experimental import pallas as pl
from jax.experimental.pallas import tpu as pltpu
from jax.experimental.pallas import tpu_sc as plsc

_N = 32768
_B = 128
_K = 256
_TEMPERATURE = 0.5

_NT = 16
_NL = _N // _NT
_NV = _NL // 16
_CAP = 64
_PK = 512
_ROWS = _B // _NT
_TOPBUF = 1152
_SENT_IDX = 0x3FFFFFFF


def _lane():
    return lax.iota(jnp.int32, 16)


def _splat(x, dtype=jnp.int32):
    return jnp.full((16,), x, dtype)


def _popcnt(mask):
    return plsc.all_reduce_population_count(mask)


def _bcast(v, i):
    idx = (_splat(0) + i).reshape(16, 1)
    return lax.gather(
        v, idx,
        dimension_numbers=lax.GatherDimensionNumbers(
            offset_dims=(), collapsed_slice_dims=(0,), start_index_map=(0,)),
        slice_sizes=(1,),
        mode=lax.GatherScatterMode.PROMISE_IN_BOUNDS)


def _sc_body(xflat, logits, gnoise, weights, out_ws, out_idx,
             keys_v, hist, hm, hidx, cand_k, cand_i, pidx, rank16,
             pk_v, pi_v, cnt_v, cntall_v, senti_v,
             idx256, w256, gidx, xg, zeros_v, rowsums_v,
             sem,
             sh_hist, sh_cnt, sh_pk, sh_pi, sh_top, sh_sums):
    wid = lax.axis_index("s")
    lanes = _lane()
    scope = jax.named_scope

    base = wid * _NL
    with scope("p1_load"):
        in_copies = [
            pltpu.async_copy(logits.at[pl.ds(base, _NL)],
                             xg.at[pl.ds(0, _NL)], sem),
            pltpu.async_copy(gnoise.at[pl.ds(base, _NL)],
                             xg.at[pl.ds(_NL, _NL)], sem),
            pltpu.async_copy(weights, w256, sem),
        ]

    def _hzero(j, _):
        for u in range(4):
            hist[pl.ds((j * 4 + u) * 16, 16)] = _splat(0)
        return 0

    lax.fori_loop(0, 65, _hzero, 0)

    def _hidx_step(j, _):
        r = j // 8
        q = j % 8
        hidx[r, pl.ds(q * 16, 16)] = _splat(r * 128 + q * 16) + lanes
        return 0

    lax.fori_loop(0, 64, _hidx_step, 0)

    def _zero_step(j, _):
        zeros_v[pl.ds(j * 16, 16)] = _splat(0)
        return 0

    lax.fori_loop(0, 16, _zero_step, 0)
    senti_v[pl.ds(0, 16)] = _splat(_SENT_IDX)
    senti_v[pl.ds(16, 16)] = _splat(_SENT_IDX)

    @pl.when(wid == 0)
    def _():
        for q in range(4):
            pltpu.sync_copy(zeros_v, sh_hist.at[pl.ds(q * 256, 256)])

    ones = _splat(1)
    with scope("p1_wait"):
        for c in in_copies:
            c.wait()

    def _keys_step(j, _):
        for u in range(2):
            o = (j * 2 + u) * 16
            y = xg[pl.ds(o, 16)] + xg[pl.ds(_NL + o, 16)]
            bits = lax.bitcast_convert_type(y, jnp.int32)
            ubits = plsc.bitcast(bits, jnp.uint32)
            key = jnp.where(bits < 0, ~ubits, ubits | jnp.uint32(0x80000000))
            keys_v[pl.ds(o, 16)] = key
            digit = (key >> jnp.uint32(24)).astype(jnp.int32)
            plsc.addupdate_scatter(hist, [lanes * 257 + digit], ones)
        return 0

    with scope("p1_keys"):
        lax.fori_loop(0, _NV // 2, _keys_step, 0)
    with scope("bar1"):
        plsc.subcore_barrier()

    prefix = _splat(0, jnp.uint32)
    rem = _splat(_K)
    for p, shift in enumerate((24, 16, 8, 0)):
        maskhi = jnp.uint32((0xFFFFFF00 << shift) & 0xFFFFFFFF)

        if p == 1:
            def _hfill1(j, _, _prefix=prefix):
                b0 = _prefix >> jnp.uint32(24)
                for u in range(2):
                    k = keys_v[pl.ds((j * 2 + u) * 16, 16)]
                    m = (k >> jnp.uint32(24)) == b0
                    digit = ((k >> jnp.uint32(16))
                             & jnp.uint32(0xFF)).astype(jnp.int32)
                    plsc.addupdate_scatter(hist, [lanes * 257 + digit],
                                           ones, mask=m)
                return 0

            with scope("p2_fill"):
                lax.fori_loop(0, _NV // 2, _hfill1, 0)
        elif p > 1:
            def _hfill(j, _, _prefix=prefix, _shift=shift, _maskhi=maskhi):
                pref = _prefix & _maskhi
                ms = []
                hit = None
                for u in range(8):
                    k = keys_v[pl.ds((j * 8 + u) * 16, 16)]
                    m = (k & _maskhi) == pref
                    ms.append(m)
                    hit = m if hit is None else (hit | m)

                @pl.when(jnp.any(hit))
                def _():
                    for u in range(8):
                        k = keys_v[pl.ds((j * 8 + u) * 16, 16)]
                        digit = ((k >> jnp.uint32(_shift))
                                 & jnp.uint32(0xFF)).astype(jnp.int32)
                        plsc.addupdate_scatter(hist, [lanes * 257 + digit],
                                               ones, mask=ms[u])
                return 0

            with scope("p2_fill"):
                lax.fori_loop(0, _NV // 8, _hfill, 0)

        def _hreduce(j, _):
            acc = hist[pl.ds(j * 16, 16)]
            hist[pl.ds(j * 16, 16)] = _splat(0)
            for l in range(1, 16):
                acc = acc + hist[pl.ds(l * 257 + j * 16, 16)]
                hist[pl.ds(l * 257 + j * 16, 16)] = _splat(0)
            hm[pl.ds(j * 16, 16)] = acc
            return 0

        with scope("p2_reduce"):
            lax.fori_loop(0, 16, _hreduce, 0)

        with scope("p2_merge"):
            pltpu.sync_copy(hm.at[pl.ds(0, 128)],
                            sh_hist.at[hidx.at[2 * p]], add=True)
            pltpu.sync_copy(hm.at[pl.ds(128, 128)],
                            sh_hist.at[hidx.at[2 * p + 1]], add=True)
        if p == 3:
            pltpu.sync_copy(zeros_v.at[pl.ds(0, 32)],
                            sh_pk.at[pl.ds(wid * 32, 32)])
            pltpu.sync_copy(senti_v, sh_pi.at[pl.ds(wid * 32, 32)])
        with scope("p2_bar"):
            plsc.subcore_barrier()
        with scope("p2_read"):
            pltpu.sync_copy(sh_hist.at[pl.ds(p * 256, 256)], hm)

        def _select(t, carry):
            B, sfxB, hB, above = carry
            vj = 15 - t
            v = hm[pl.ds(vj * 16, 16)]
            sfx = lax.rev(plsc.cumsum(lax.rev(v, (0,))), (0,)) + above
            cond = sfx >= rem
            cnt = _popcnt(cond)
            first = (B < 0) & (cnt > 0)
            this_b = _splat(vj * 16) + cnt - 1
            sel = jnp.maximum(cnt - 1, 0)
            B = jnp.where(first, this_b, B)
            hB = jnp.where(first, _bcast(v, sel), hB)
            sfxB = jnp.where(first, _bcast(sfx, sel), sfxB)
            above = _bcast(sfx, _splat(0))
            return B, sfxB, hB, above

        with scope("p2_select"):
            B, sfxB, hB, _ = lax.fori_loop(
                0, 16, _select,
                (_splat(-1), _splat(0), _splat(0), _splat(0)))
        rem = rem - (sfxB - hB)
        prefix = prefix | (B.astype(jnp.uint32) << jnp.uint32(shift))

    def _compact(j, cnt):
        kms = []
        hit = None
        for u in range(8):
            k = keys_v[pl.ds((j * 8 + u) * 16, 16)]
            m = k >= prefix
            kms.append((k, m))
            hit = m if hit is None else (hit | m)

        @pl.when(jnp.any(hit))
        def _():
            c = cnt
            for u in range(8):
                k, m = kms[u]
                pos = c + plsc.cumsum(jnp.where(m, 1, 0)) - 1
                pos = jnp.minimum(pos, _CAP - 1)
                plsc.store_scatter(cand_k, [pos],
                                   plsc.bitcast(k, jnp.int32), mask=m)
                gi = _splat((j * 8 + u) * 16) + lanes + base
                plsc.store_scatter(cand_i, [pos], gi, mask=m)
                c = c + _popcnt(m)

        for u in range(8):
            cnt = cnt + _popcnt(kms[u][1])
        return cnt

    with scope("p3_compact"):
        cnt = lax.fori_loop(0, _NV // 8, _compact, _splat(0))
    cnt = jnp.minimum(cnt, _CAP)

    cnt_v[...] = cnt
    with scope("p3_cntx"):
        pltpu.sync_copy(cnt_v, sh_cnt.at[pl.ds(wid * 16, 16)])
        plsc.subcore_barrier()
        pltpu.sync_copy(sh_cnt, cntall_v)

    pbase = _splat(0)
    ntot = _splat(0)
    for t in range(_NT):
        v = cntall_v[pl.ds(t * 16, 16)]
        pbase = pbase + jnp.where(_splat(t) < wid, v, 0)
        ntot = ntot + v

    def _pidx_step(q, _):
        slot = _splat(q * 16) + lanes
        pos = jnp.where(slot < cnt, pbase + slot, _splat(_PK) + wid)
        pidx[pl.ds(q * 16, 16)] = jnp.minimum(pos, _PK + 63)
        return 0

    with scope("p3_pack"):
        lax.fori_loop(0, _CAP // 16, _pidx_step, 0)
        pltpu.sync_copy(cand_k, sh_pk.at[pidx])
        pltpu.sync_copy(cand_i, sh_pi.at[pidx])
        plsc.subcore_barrier()
        pltpu.sync_copy(sh_pk.at[pl.ds(0, _PK)], pk_v)
        pltpu.sync_copy(sh_pi.at[pl.ds(0, _PK)], pi_v)

    ntot_s = ntot[0]

    with scope("p3_rank"):
        for rnd in range(2):
            vsel = wid if rnd == 0 else 16 + (15 - wid)

            @pl.when(ntot_s > vsel * 16)
            def _():
                ck = plsc.bitcast(pk_v[pl.ds(vsel * 16, 16)], jnp.uint32)
                ci = pi_v[pl.ds(vsel * 16, 16)]

                def _rank(e, r):
                    ke = plsc.bitcast(
                        plsc.load_gather(pk_v, [_splat(0) + e]), jnp.uint32)
                    ie = plsc.load_gather(pi_v, [_splat(0) + e])
                    gt = ke > ck
                    tie = (ke == ck) & (ie < ci)
                    return r + gt.astype(jnp.int32) + tie.astype(jnp.int32)

                r = lax.fori_loop(0, ntot_s, _rank, _splat(0))
                rank16[...] = jnp.minimum(r, _TOPBUF - 1)
                pltpu.sync_copy(pi_v.at[pl.ds(vsel * 16, 16)],
                                sh_top.at[rank16])

    with scope("p3_rankbar"):
        plsc.subcore_barrier()

    with scope("p4_idx"):
        pltpu.sync_copy(sh_top.at[pl.ds(0, _K)], idx256)

    def _cphys_step(q, _):
        cj = idx256[pl.ds(q * 16, 16)]
        idx256[pl.ds(q * 16, 16)] = ((cj >> 7) << 10) | (cj & 127)
        return 0

    def _gfill(j, _):
        r = j // 16
        q = j % 16
        b = wid * _ROWS + r
        rb = (b // 8) * (_N * 8) + (b % 8) * 128
        gidx[pl.ds(j * 16, 16)] = idx256[pl.ds(q * 16, 16)] + rb
        return 0

    with scope("p4_gidx"):
        lax.fori_loop(0, _K // 16, _cphys_step, 0)
        lax.fori_loop(0, _ROWS * 16, _gfill, 0)

    copies = []
    for r in range(_ROWS):
        for h in range(2):
            o = r * _K + h * 128
            copies.append(pltpu.async_copy(
                xflat.at[gidx.at[pl.ds(o, 128)]],
                xg.at[pl.ds(o, 128)], sem))
    with scope("p4_gather"):
        for c in copies:
            c.wait()

    def _wsum(r, sums):
        def _acc(q, a):
            a0 = a + xg[pl.ds(r * _K + q * 32, 16)] * w256[pl.ds(q * 32, 16)]
            return a0 + (xg[pl.ds(r * _K + q * 32 + 16, 16)]
                         * w256[pl.ds(q * 32 + 16, 16)])
        acc = lax.fori_loop(0, _K // 32, _acc, jnp.zeros((16,), jnp.float32))
        s = jnp.sum(acc)
        return jnp.where(lanes == r, jnp.full((16,), s, jnp.float32), sums)

    with scope("p4_wsum"):
        sums = lax.fori_loop(0, _ROWS, _wsum, jnp.zeros((16,), jnp.float32))
    rowsums_v[...] = sums
    with scope("p4_out"):
        pltpu.sync_copy(rowsums_v.at[pl.ds(0, _ROWS)],
                        sh_sums.at[pl.ds(wid * _ROWS, _ROWS)])
        plsc.subcore_barrier()

    @pl.when(wid == 0)
    def _():
        pltpu.sync_copy(sh_sums, out_ws)
        pltpu.sync_copy(sh_top.at[pl.ds(0, _K)], out_idx)


@functools.partial(jax.jit, static_argnums=())
def _run(xflat, logits, gnoise, weights):
    mesh = plsc.VectorSubcoreMesh(
        core_axis_name="c", subcore_axis_name="s", num_cores=1)
    kern = pl.kernel(
        _sc_body,
        out_type=(
            jax.ShapeDtypeStruct((_B,), jnp.float32),
            jax.ShapeDtypeStruct((_K,), jnp.int32),
        ),
        mesh=mesh,
        compiler_params=pltpu.CompilerParams(needs_layout_passes=False),
        scratch_types=[
            pltpu.VMEM((_NL,), jnp.uint32),
            pltpu.VMEM((16 * 257 + 15,), jnp.int32),
            pltpu.VMEM((256,), jnp.int32),
            pltpu.VMEM((8, 128), jnp.int32),
            pltpu.VMEM((_CAP,), jnp.int32),
            pltpu.VMEM((_CAP,), jnp.int32),
            pltpu.VMEM((_CAP,), jnp.int32),
            pltpu.VMEM((16,), jnp.int32),
            pltpu.VMEM((_PK,), jnp.int32),
            pltpu.VMEM((_PK,), jnp.int32),
            pltpu.VMEM((16,), jnp.int32),
            pltpu.VMEM((256,), jnp.int32),
            pltpu.VMEM((32,), jnp.int32),
            pltpu.VMEM((_K,), jnp.int32),
            pltpu.VMEM((_K,), jnp.float32),
            pltpu.VMEM((_ROWS * _K,), jnp.int32),
            pltpu.VMEM((2 * _NL,), jnp.float32),
            pltpu.VMEM((256,), jnp.int32),
            pltpu.VMEM((16,), jnp.float32),
            pltpu.SemaphoreType.DMA,
            pltpu.VMEM_SHARED((1024,), jnp.int32),
            pltpu.VMEM_SHARED((256,), jnp.int32),
            pltpu.VMEM_SHARED((_PK + 64,), jnp.int32),
            pltpu.VMEM_SHARED((_PK + 64,), jnp.int32),
            pltpu.VMEM_SHARED((_TOPBUF,), jnp.int32),
            pltpu.VMEM_SHARED((_B,), jnp.float32),
        ],
    )
    return kern(xflat, logits, gnoise, weights)


def kernel(x, logits, output_weights):
    gkey = jax.random.key(42)
    u = jax.random.uniform(gkey, (1, _N), dtype=jnp.float32)
    eps = 1e-20
    g = (-jnp.log(-jnp.log(u + eps) + eps)).reshape(-1)
    xp = x.reshape(16, 8, 256, 128).transpose(0, 2, 1, 3).reshape(-1)
    ws, idx = _run(xp, logits, g, output_weights)
    return (ws.reshape(_B, 1), idx.reshape(1, _K), output_weights)


# speedup vs baseline: 1.3641x; 1.0581x over previous
import functools

import jax
import jax.numpy as jnp
from jax import lax
from jax.experimental import pallas as pl
from jax.experimental.pallas import tpu as pltpu
from jax.experimental.pallas import tpu_sc as plsc

_N = 32768
_B = 128
_K = 256
_TEMPERATURE = 0.5

_NT = 16
_NL = _N // _NT
_NV = _NL // 16
_CAP = 64
_PK = 512
_ROWS = _B // _NT
_TOPBUF = 1152
_SENT_IDX = 0x3FFFFFFF


def _lane():
    return lax.iota(jnp.int32, 16)


def _splat(x, dtype=jnp.int32):
    return jnp.full((16,), x, dtype)


def _popcnt(mask):
    return plsc.all_reduce_population_count(mask)


def _bcast(v, i):
    idx = (_splat(0) + i).reshape(16, 1)
    return lax.gather(
        v, idx,
        dimension_numbers=lax.GatherDimensionNumbers(
            offset_dims=(), collapsed_slice_dims=(0,), start_index_map=(0,)),
        slice_sizes=(1,),
        mode=lax.GatherScatterMode.PROMISE_IN_BOUNDS)


def _sc_body(xflat, logits, gnoise, weights, out_ws, out_idx,
             keys_v, bk_v, bi_v, hist, hm, hidx, cand_k, cand_i, pidx, rank16,
             pk_v, pi_v, cnt_v, cntall_v, senti_v,
             idx256, w256, gidx, xg, zeros_v, rowsums_v,
             sem,
             sh_hist, sh_cnt, sh_pk, sh_pi, sh_top, sh_sums):
    wid = lax.axis_index("s")
    lanes = _lane()
    scope = jax.named_scope

    base = wid * _NL
    with scope("p1_load"):
        in_copies = [
            pltpu.async_copy(logits.at[pl.ds(base, _NL)],
                             xg.at[pl.ds(0, _NL)], sem),
            pltpu.async_copy(gnoise.at[pl.ds(base, _NL)],
                             xg.at[pl.ds(_NL, _NL)], sem),
            pltpu.async_copy(weights, w256, sem),
        ]

    def _hzero(j, _):
        for u in range(4):
            hist[pl.ds((j * 4 + u) * 16, 16)] = _splat(0)
        return 0

    lax.fori_loop(0, 65, _hzero, 0)

    def _hidx_step(j, _):
        r = j // 8
        q = j % 8
        hidx[r, pl.ds(q * 16, 16)] = _splat(r * 128 + q * 16) + lanes
        return 0

    lax.fori_loop(0, 64, _hidx_step, 0)

    def _zero_step(j, _):
        zeros_v[pl.ds(j * 16, 16)] = _splat(0)
        return 0

    lax.fori_loop(0, 16, _zero_step, 0)
    senti_v[pl.ds(0, 16)] = _splat(_SENT_IDX)
    senti_v[pl.ds(16, 16)] = _splat(_SENT_IDX)

    @pl.when(wid == 0)
    def _():
        for q in range(4):
            pltpu.sync_copy(zeros_v, sh_hist.at[pl.ds(q * 256, 256)])

    ones = _splat(1)
    with scope("p1_wait"):
        for c in in_copies:
            c.wait()

    def _keys_step(j, _):
        for u in range(2):
            o = (j * 2 + u) * 16
            y = xg[pl.ds(o, 16)] + xg[pl.ds(_NL + o, 16)]
            bits = lax.bitcast_convert_type(y, jnp.int32)
            ubits = plsc.bitcast(bits, jnp.uint32)
            key = jnp.where(bits < 0, ~ubits, ubits | jnp.uint32(0x80000000))
            keys_v[pl.ds(o, 16)] = key
            digit = (key >> jnp.uint32(24)).astype(jnp.int32)
            plsc.addupdate_scatter(hist, [lanes * 257 + digit], ones)
        return 0

    with scope("p1_keys"):
        lax.fori_loop(0, _NV // 2, _keys_step, 0)
    with scope("bar1"):
        plsc.subcore_barrier()

    prefix = _splat(0, jnp.uint32)
    rem = _splat(_K)
    for p, shift in enumerate((24, 16, 8, 0)):
        maskhi = jnp.uint32((0xFFFFFF00 << shift) & 0xFFFFFFFF)

        if p > 0:
            def _hfill(j, _, _prefix=prefix, _shift=shift, _maskhi=maskhi):
                k = bk_v[pl.ds(j * 16, 16)]
                match = (k & _maskhi) == (_prefix & _maskhi)
                digit = ((k >> jnp.uint32(_shift))
                         & jnp.uint32(0xFF)).astype(jnp.int32)
                plsc.addupdate_scatter(hist, [lanes * 257 + digit],
                                       ones, mask=match)
                return 0

            with scope("p2_fill"):
                lax.fori_loop(0, nb_s, _hfill, 0)

        def _hreduce(j, _):
            acc = hist[pl.ds(j * 16, 16)]
            hist[pl.ds(j * 16, 16)] = _splat(0)
            for l in range(1, 16):
                acc = acc + hist[pl.ds(l * 257 + j * 16, 16)]
                hist[pl.ds(l * 257 + j * 16, 16)] = _splat(0)
            hm[pl.ds(j * 16, 16)] = acc
            return 0

        with scope("p2_reduce"):
            lax.fori_loop(0, 16, _hreduce, 0)

        with scope("p2_merge"):
            pltpu.sync_copy(hm.at[pl.ds(0, 128)],
                            sh_hist.at[hidx.at[2 * p]], add=True)
            pltpu.sync_copy(hm.at[pl.ds(128, 128)],
                            sh_hist.at[hidx.at[2 * p + 1]], add=True)
        if p == 3:
            pltpu.sync_copy(zeros_v.at[pl.ds(0, 32)],
                            sh_pk.at[pl.ds(wid * 32, 32)])
            pltpu.sync_copy(senti_v, sh_pi.at[pl.ds(wid * 32, 32)])
        with scope("p2_bar"):
            plsc.subcore_barrier()
        with scope("p2_read"):
            pltpu.sync_copy(sh_hist.at[pl.ds(p * 256, 256)], hm)

        def _select(t, carry):
            B, sfxB, hB, above = carry
            vj = 15 - t
            v = hm[pl.ds(vj * 16, 16)]
            sfx = lax.rev(plsc.cumsum(lax.rev(v, (0,))), (0,)) + above
            cond = sfx >= rem
            cnt = _popcnt(cond)
            first = (B < 0) & (cnt > 0)
            this_b = _splat(vj * 16) + cnt - 1
            sel = jnp.maximum(cnt - 1, 0)
            B = jnp.where(first, this_b, B)
            hB = jnp.where(first, _bcast(v, sel), hB)
            sfxB = jnp.where(first, _bcast(sfx, sel), sfxB)
            above = _bcast(sfx, _splat(0))
            return B, sfxB, hB, above

        with scope("p2_select"):
            B, sfxB, hB, _ = lax.fori_loop(
                0, 16, _select,
                (_splat(-1), _splat(0), _splat(0), _splat(0)))
        rem = rem - (sfxB - hB)
        prefix = prefix | (B.astype(jnp.uint32) << jnp.uint32(shift))

        if p == 0:
            b0 = prefix >> jnp.uint32(24)

            def _bcompact(j, boff):
                k = keys_v[pl.ds(j * 16, 16)]
                m = (k >> jnp.uint32(24)) >= b0
                plsc.store_compressed(bk_v.at[pl.ds(boff, 16)], k, mask=m)
                gi = _splat(j * 16) + lanes + base
                plsc.store_compressed(bi_v.at[pl.ds(boff, 16)], gi, mask=m)
                return boff + _popcnt(m)[0]

            with scope("p2_bcompact"):
                boff_s = lax.fori_loop(0, _NV, _bcompact, wid * 0)
                bk_v[pl.ds(boff_s, 16)] = _splat(0, jnp.uint32)
            nb_s = (boff_s + 15) // 16

    def _compact(j, cnt):
        k = bk_v[pl.ds(j * 16, 16)]
        m = k >= prefix
        pos = cnt + plsc.cumsum(jnp.where(m, 1, 0)) - 1
        pos = jnp.minimum(pos, _CAP - 1)
        plsc.store_scatter(cand_k, [pos],
                           plsc.bitcast(k, jnp.int32), mask=m)
        gi = bi_v[pl.ds(j * 16, 16)]
        plsc.store_scatter(cand_i, [pos], gi, mask=m)
        return cnt + _popcnt(m)

    with scope("p3_compact"):
        cnt = lax.fori_loop(0, nb_s, _compact, _splat(0))
    cnt = jnp.minimum(cnt, _CAP)

    cnt_v[...] = cnt
    with scope("p3_cntx"):
        pltpu.sync_copy(cnt_v, sh_cnt.at[pl.ds(wid * 16, 16)])
        plsc.subcore_barrier()
        pltpu.sync_copy(sh_cnt, cntall_v)

    pbase = _splat(0)
    ntot = _splat(0)
    for t in range(_NT):
        v = cntall_v[pl.ds(t * 16, 16)]
        pbase = pbase + jnp.where(_splat(t) < wid, v, 0)
        ntot = ntot + v

    def _pidx_step(q, _):
        slot = _splat(q * 16) + lanes
        pos = jnp.where(slot < cnt, pbase + slot, _splat(_PK) + wid)
        pidx[pl.ds(q * 16, 16)] = jnp.minimum(pos, _PK + 63)
        return 0

    with scope("p3_pack"):
        lax.fori_loop(0, _CAP // 16, _pidx_step, 0)
        pltpu.sync_copy(cand_k, sh_pk.at[pidx])
        pltpu.sync_copy(cand_i, sh_pi.at[pidx])
        plsc.subcore_barrier()
        pltpu.sync_copy(sh_pk.at[pl.ds(0, _PK)], pk_v)
        pltpu.sync_copy(sh_pi.at[pl.ds(0, _PK)], pi_v)

    ntot_s = ntot[0]

    with scope("p3_rank"):
        for rnd in range(2):
            vsel = wid if rnd == 0 else 16 + (15 - wid)

            @pl.when(ntot_s > vsel * 16)
            def _():
                ck = plsc.bitcast(pk_v[pl.ds(vsel * 16, 16)], jnp.uint32)
                ci = pi_v[pl.ds(vsel * 16, 16)]

                def _rank(e, r):
                    ke = plsc.bitcast(
                        plsc.load_gather(pk_v, [_splat(0) + e]), jnp.uint32)
                    ie = plsc.load_gather(pi_v, [_splat(0) + e])
                    gt = ke > ck
                    tie = (ke == ck) & (ie < ci)
                    return r + gt.astype(jnp.int32) + tie.astype(jnp.int32)

                r = lax.fori_loop(0, ntot_s, _rank, _splat(0))
                rank16[...] = jnp.minimum(r, _TOPBUF - 1)
                pltpu.sync_copy(pi_v.at[pl.ds(vsel * 16, 16)],
                                sh_top.at[rank16])

    with scope("p3_rankbar"):
        plsc.subcore_barrier()

    with scope("p4_idx"):
        pltpu.sync_copy(sh_top.at[pl.ds(0, _K)], idx256)

    def _cphys_step(q, _):
        cj = idx256[pl.ds(q * 16, 16)]
        idx256[pl.ds(q * 16, 16)] = ((cj >> 7) << 10) | (cj & 127)
        return 0

    def _gfill(j, _):
        r = j // 16
        q = j % 16
        b = wid * _ROWS + r
        rb = (b // 8) * (_N * 8) + (b % 8) * 128
        gidx[pl.ds(j * 16, 16)] = idx256[pl.ds(q * 16, 16)] + rb
        return 0

    with scope("p4_gidx"):
        lax.fori_loop(0, _K // 16, _cphys_step, 0)
        lax.fori_loop(0, _ROWS * 16, _gfill, 0)

    copies = []
    for r in range(_ROWS):
        for h in range(2):
            o = r * _K + h * 128
            copies.append(pltpu.async_copy(
                xflat.at[gidx.at[pl.ds(o, 128)]],
                xg.at[pl.ds(o, 128)], sem))
    with scope("p4_gather"):
        for c in copies:
            c.wait()

    def _wsum(r, sums):
        def _acc(q, a):
            a0 = a + xg[pl.ds(r * _K + q * 32, 16)] * w256[pl.ds(q * 32, 16)]
            return a0 + (xg[pl.ds(r * _K + q * 32 + 16, 16)]
                         * w256[pl.ds(q * 32 + 16, 16)])
        acc = lax.fori_loop(0, _K // 32, _acc, jnp.zeros((16,), jnp.float32))
        s = jnp.sum(acc)
        return jnp.where(lanes == r, jnp.full((16,), s, jnp.float32), sums)

    with scope("p4_wsum"):
        sums = lax.fori_loop(0, _ROWS, _wsum, jnp.zeros((16,), jnp.float32))
    rowsums_v[...] = sums
    with scope("p4_out"):
        pltpu.sync_copy(rowsums_v.at[pl.ds(0, _ROWS)],
                        sh_sums.at[pl.ds(wid * _ROWS, _ROWS)])
        plsc.subcore_barrier()

    @pl.when(wid == 0)
    def _():
        pltpu.sync_copy(sh_sums, out_ws)
        pltpu.sync_copy(sh_top.at[pl.ds(0, _K)], out_idx)


@functools.partial(jax.jit, static_argnums=())
def _run(xflat, logits, gnoise, weights):
    mesh = plsc.VectorSubcoreMesh(
        core_axis_name="c", subcore_axis_name="s", num_cores=1)
    kern = pl.kernel(
        _sc_body,
        out_type=(
            jax.ShapeDtypeStruct((_B,), jnp.float32),
            jax.ShapeDtypeStruct((_K,), jnp.int32),
        ),
        mesh=mesh,
        compiler_params=pltpu.CompilerParams(needs_layout_passes=False),
        scratch_types=[
            pltpu.VMEM((_NL,), jnp.uint32),
            pltpu.VMEM((_NL + 16,), jnp.uint32),
            pltpu.VMEM((_NL + 16,), jnp.int32),
            pltpu.VMEM((16 * 257 + 15,), jnp.int32),
            pltpu.VMEM((256,), jnp.int32),
            pltpu.VMEM((8, 128), jnp.int32),
            pltpu.VMEM((_CAP,), jnp.int32),
            pltpu.VMEM((_CAP,), jnp.int32),
            pltpu.VMEM((_CAP,), jnp.int32),
            pltpu.VMEM((16,), jnp.int32),
            pltpu.VMEM((_PK,), jnp.int32),
            pltpu.VMEM((_PK,), jnp.int32),
            pltpu.VMEM((16,), jnp.int32),
            pltpu.VMEM((256,), jnp.int32),
            pltpu.VMEM((32,), jnp.int32),
            pltpu.VMEM((_K,), jnp.int32),
            pltpu.VMEM((_K,), jnp.float32),
            pltpu.VMEM((_ROWS * _K,), jnp.int32),
            pltpu.VMEM((2 * _NL,), jnp.float32),
            pltpu.VMEM((256,), jnp.int32),
            pltpu.VMEM((16,), jnp.float32),
            pltpu.SemaphoreType.DMA,
            pltpu.VMEM_SHARED((1024,), jnp.int32),
            pltpu.VMEM_SHARED((256,), jnp.int32),
            pltpu.VMEM_SHARED((_PK + 64,), jnp.int32),
            pltpu.VMEM_SHARED((_PK + 64,), jnp.int32),
            pltpu.VMEM_SHARED((_TOPBUF,), jnp.int32),
            pltpu.VMEM_SHARED((_B,), jnp.float32),
        ],
    )
    return kern(xflat, logits, gnoise, weights)


def kernel(x, logits, output_weights):
    gkey = jax.random.key(42)
    u = jax.random.uniform(gkey, (1, _N), dtype=jnp.float32)
    eps = 1e-20
    g = (-jnp.log(-jnp.log(u + eps) + eps)).reshape(-1)
    xp = x.reshape(16, 8, 256, 128).transpose(0, 2, 1, 3).reshape(-1)
    ws, idx = _run(xp, logits, g, output_weights)
    return (ws.reshape(_B, 1), idx.reshape(1, _K), output_weights)
